# baseline (speedup 1.0000x reference)
import jax
import jax.numpy as jnp
from jax import lax
from jax.experimental import pallas as pl
from jax.experimental.pallas import tpu as pltpu
from jax.experimental.pallas import tpu_sc as plsc

SEQ = 2048
EMB = 32
TMPL = 2 * SEQ
NC, NS = 2, 16
NW = NC * NS
ROWS_PER_W = SEQ // NW
TROWS_PER_S = TMPL // NS
GCHUNK = 128


def _sc_body(idx_h, table_h, out_h, idx_v, rows_v, tmpl_sh, sem):
    c = lax.axis_index("c")
    s = lax.axis_index("s")

    for chunk in range(TROWS_PER_S // GCHUNK):
        base = s * TROWS_PER_S + chunk * GCHUNK
        pltpu.sync_copy(idx_h.at[pl.ds(base, GCHUNK)], idx_v)
        pltpu.async_copy(table_h.at[idx_v], rows_v, sem).wait()
        pltpu.sync_copy(rows_v, tmpl_sh.at[pl.ds(base, GCHUNK)])
    plsc.subcore_barrier()

    wid = s * NC + c

    def fire_row(r, carry):
        i = wid * ROWS_PER_W + r
        start = (SEQ - 1) - i
        pltpu.async_copy(tmpl_sh.at[pl.ds(start, SEQ)], out_h.at[i], sem)
        return carry

    lax.fori_loop(0, ROWS_PER_W, fire_row, 0)

    def drain_row(r, carry):
        pltpu.make_async_copy(tmpl_sh.at[pl.ds(0, SEQ)], out_h.at[0], sem).wait()
        return carry

    lax.fori_loop(0, ROWS_PER_W, drain_row, 0)


_sc_call = pl.kernel(
    _sc_body,
    out_type=jax.ShapeDtypeStruct((SEQ, SEQ, EMB), jnp.float32),
    mesh=plsc.VectorSubcoreMesh(core_axis_name="c", subcore_axis_name="s"),
    scratch_types=[
        pltpu.VMEM((GCHUNK,), jnp.int32),
        pltpu.VMEM((GCHUNK, EMB), jnp.float32),
        pltpu.VMEM_SHARED((TMPL, EMB), jnp.float32),
        pltpu.SemaphoreType.DMA,
    ],
    compiler_params=pltpu.CompilerParams(use_tc_tiling_on_sc=False),
)


def kernel(table, seq_len, layer_attention_span):
    span = jnp.asarray(layer_attention_span, jnp.int32)
    k = jnp.arange(TMPL, dtype=jnp.int32)
    idx = jnp.clip(jnp.abs(k - (SEQ - 1)), 0, span)
    return _sc_call(idx, table)


# speedup vs baseline: 13.8669x; 13.8669x over previous
import jax
import jax.numpy as jnp
from jax.experimental import pallas as pl
from jax.experimental.pallas import tpu as pltpu

SEQ = 2048
EMB = 32
VOCAB = 129
TW = 2 * SEQ
NB = 128


def _tc_body(span_ref, tablet_ref, out_ref):
    g = pl.program_id(0)
    s0 = (NB - 1) - g
    span = span_ref[0]

    vv = jax.lax.broadcasted_iota(jnp.int32, (VOCAB, TW), 0)
    kk = jax.lax.broadcasted_iota(jnp.int32, (VOCAB, TW), 1) + (s0 - (SEQ - 1))
    idx = jnp.clip(jnp.abs(kk), 0, span)
    oh = (vv == idx).astype(jnp.float32)
    t_s0 = jnp.dot(tablet_ref[...], oh, preferred_element_type=jnp.float32)

    for a in range(SEQ // NB):
        off = NB * (SEQ // NB - 1 - a)
        out_ref[a, 0] = t_s0[:, off:off + SEQ]


def kernel(table, seq_len, layer_attention_span):
    span = jnp.asarray(layer_attention_span, jnp.int32).reshape(1)
    tablet = table.T

    out4 = pl.pallas_call(
        _tc_body,
        grid=(NB,),
        in_specs=[
            pl.BlockSpec(memory_space=pltpu.SMEM),
            pl.BlockSpec((EMB, VOCAB), lambda g: (0, 0)),
        ],
        out_specs=pl.BlockSpec((SEQ // NB, 1, EMB, SEQ), lambda g: (0, g, 0, 0)),
        out_shape=jax.ShapeDtypeStruct((SEQ // NB, NB, EMB, SEQ), jnp.float32),
        compiler_params=pltpu.CompilerParams(
            dimension_semantics=("arbitrary",),
        ),
    )(span, tablet)

    out_phys = out4.reshape(SEQ, EMB, SEQ)
    return jnp.transpose(out_phys, (0, 2, 1))
